# Initial kernel scaffold; baseline (speedup 1.0000x reference)
import jax
import jax.numpy as jnp
from jax.experimental import pallas as pl


def kernel(feature, edge_index, W1, b1, W2, b2, W3, b3):
    raise NotImplementedError("write your pallas kernel here")


# speedup vs baseline: 10.0227x; 10.0227x over previous
import functools

import jax
import jax.numpy as jnp
from jax import lax
from jax.experimental import pallas as pl
from jax.experimental.pallas import tpu as pltpu
from jax.experimental.pallas import tpu_sc as plsc

_N = 10000
_E = 320000
_D_IN = 128
_D_HID = 64
_D_OUT = 128

_NC = 2
_NS = 16
_NW = _NC * _NS
_EPW = _E // _NW
_CHUNK = 125
_NCHUNK = _EPW // _CHUNK
_ROWS_PT = _N // _NS


def _mm1_body(x_ref, w_ref, o_ref):
    o_ref[...] = jnp.dot(x_ref[...], w_ref[...], preferred_element_type=jnp.float32)


def _mlp_body(p0_ref, p1_ref, b1_ref, w2_ref, b2_ref, w3_ref, b3_ref, o_ref):
    h = jnp.maximum(p0_ref[...] + p1_ref[...] + b1_ref[...], 0.0)
    h = jnp.maximum(
        jnp.dot(h, w2_ref[...], preferred_element_type=jnp.float32) + b2_ref[...], 0.0)
    o_ref[...] = jnp.dot(h, w3_ref[...], preferred_element_type=jnp.float32) + b3_ref[...]


def _sc_aggregate(f1, src3, dst3):
    mesh = plsc.VectorSubcoreMesh(core_axis_name="c", subcore_axis_name="s")

    @functools.partial(
        pl.kernel,
        out_type=jax.ShapeDtypeStruct((_NC, _N, _D_HID), jnp.float32),
        mesh=mesh,
        scratch_types=[
            pltpu.VMEM((_NCHUNK, _CHUNK), jnp.int32),
            pltpu.VMEM((_NCHUNK, _CHUNK), jnp.int32),
            pltpu.VMEM((_CHUNK, _D_HID), jnp.float32),
            pltpu.VMEM_SHARED((_N, _D_HID), jnp.float32),
            pltpu.SemaphoreType.DMA,
        ],
        compiler_params=pltpu.CompilerParams(use_tc_tiling_on_sc=False),
    )
    def body(f1_hbm, src_hbm, dst_hbm, out_hbm, src_v, dst_v, rows_v, acc_sh, sem):
        c = lax.axis_index("c")
        s = lax.axis_index("s")
        wid = s * _NC + c

        pltpu.sync_copy(src_hbm.at[wid], src_v)
        pltpu.sync_copy(dst_hbm.at[wid], dst_v)

        def zero_body(i, carry):
            for k in range(_D_HID // 16):
                rows_v[i, pl.ds(k * 16, 16)] = jnp.zeros((16,), jnp.float32)
            return carry
        lax.fori_loop(0, _CHUNK, zero_body, 0)
        slab = s * _ROWS_PT
        for r in range(_ROWS_PT // _CHUNK):
            pltpu.sync_copy(rows_v, acc_sh.at[pl.ds(slab + r * _CHUNK, _CHUNK)])
        plsc.subcore_barrier()

        def edge_body(j, carry):
            pltpu.async_copy(f1_hbm.at[src_v.at[j]], rows_v, sem).wait()
            pltpu.sync_copy(rows_v, acc_sh.at[dst_v.at[j]], add=True)
            return carry
        lax.fori_loop(0, _NCHUNK, edge_body, 0)
        plsc.subcore_barrier()

        @pl.when(s < 10)
        def _():
            pltpu.sync_copy(acc_sh.at[pl.ds(s * 1000, 1000)],
                            out_hbm.at[c, pl.ds(s * 1000, 1000)])

    return body(f1, src3, dst3)


@jax.jit
def _impl(feature, edge_index, W1, b1, W2, b2, W3, b3):
    src3 = edge_index[0].reshape(_NW, _NCHUNK, _CHUNK)
    dst3 = edge_index[1].reshape(_NW, _NCHUNK, _CHUNK)

    blk = 1000
    grid = (_N // blk,)
    f1 = pl.pallas_call(
        _mm1_body,
        grid=grid,
        in_specs=[
            pl.BlockSpec((blk, _D_IN), lambda i: (i, 0)),
            pl.BlockSpec((_D_IN, _D_HID), lambda i: (0, 0)),
        ],
        out_specs=pl.BlockSpec((blk, _D_HID), lambda i: (i, 0)),
        out_shape=jax.ShapeDtypeStruct((_N, _D_HID), jnp.float32),
    )(feature, W1)

    partials = _sc_aggregate(f1, src3, dst3)

    out = pl.pallas_call(
        _mlp_body,
        grid=grid,
        in_specs=[
            pl.BlockSpec((blk, _D_HID), lambda i: (i, 0)),
            pl.BlockSpec((blk, _D_HID), lambda i: (i, 0)),
            pl.BlockSpec((1, _D_HID), lambda i: (0, 0)),
            pl.BlockSpec((_D_HID, _D_HID), lambda i: (0, 0)),
            pl.BlockSpec((1, _D_HID), lambda i: (0, 0)),
            pl.BlockSpec((_D_HID, _D_OUT), lambda i: (0, 0)),
            pl.BlockSpec((1, _D_OUT), lambda i: (0, 0)),
        ],
        out_specs=pl.BlockSpec((blk, _D_OUT), lambda i: (i, 0)),
        out_shape=jax.ShapeDtypeStruct((_N, _D_OUT), jnp.float32),
    )(partials[0], partials[1], b1.reshape(1, _D_HID), W2,
      b2.reshape(1, _D_HID), W3, b3.reshape(1, _D_OUT))
    return out


def kernel(feature, edge_index, W1, b1, W2, b2, W3, b3):
    return _impl(feature, edge_index, W1, b1, W2, b2, W3, b3)


# speedup vs baseline: 12.8989x; 1.2870x over previous
import functools

import jax
import jax.numpy as jnp
from jax import lax
from jax.experimental import pallas as pl
from jax.experimental.pallas import tpu as pltpu
from jax.experimental.pallas import tpu_sc as plsc

_N = 10000
_E = 320000
_D_IN = 128
_D_HID = 64
_D_OUT = 128

_NC = 2
_NS = 16
_NW = _NC * _NS
_EPW = _E // _NW
_CHUNK = 125
_NCHUNK = _EPW // _CHUNK
_ROWS_PT = _N // _NS
_K = 8


def _mm1_body(x_ref, w_ref, o_ref):
    o_ref[...] = jnp.dot(x_ref[...], w_ref[...], preferred_element_type=jnp.float32)


def _mlp_body(p0_ref, p1_ref, b1_ref, w2_ref, b2_ref, w3_ref, b3_ref, o_ref):
    h = jnp.maximum(p0_ref[...] + p1_ref[...] + b1_ref[...], 0.0)
    h = jnp.maximum(
        jnp.dot(h, w2_ref[...], preferred_element_type=jnp.float32) + b2_ref[...], 0.0)
    o_ref[...] = jnp.dot(h, w3_ref[...], preferred_element_type=jnp.float32) + b3_ref[...]


def _sc_aggregate(f1, src3, dst3):
    mesh = plsc.VectorSubcoreMesh(core_axis_name="c", subcore_axis_name="s")

    @functools.partial(
        pl.kernel,
        out_type=jax.ShapeDtypeStruct((_NC, _N, _D_HID), jnp.float32),
        mesh=mesh,
        scratch_types=[
            pltpu.VMEM((_NCHUNK, _CHUNK), jnp.int32),
            pltpu.VMEM((_NCHUNK, _CHUNK), jnp.int32),
            pltpu.VMEM((_K * _CHUNK, _D_HID), jnp.float32),
            pltpu.VMEM_SHARED((_N, _D_HID), jnp.float32),
            pltpu.SemaphoreType.DMA((_K,)),
        ],
        compiler_params=pltpu.CompilerParams(use_tc_tiling_on_sc=False),
    )
    def body(f1_hbm, src_hbm, dst_hbm, out_hbm, src_v, dst_v, rows_v, acc_sh, sem):
        c = lax.axis_index("c")
        s = lax.axis_index("s")
        wid = s * _NC + c

        pltpu.sync_copy(src_hbm.at[wid], src_v)
        pltpu.sync_copy(dst_hbm.at[wid], dst_v)

        def zero_body(i, carry):
            for k in range(_D_HID // 16):
                rows_v[i, pl.ds(k * 16, 16)] = jnp.zeros((16,), jnp.float32)
            return carry
        lax.fori_loop(0, _CHUNK, zero_body, 0)
        slab = s * _ROWS_PT
        for r in range(_ROWS_PT // _CHUNK):
            pltpu.sync_copy(rows_v.at[pl.ds(0, _CHUNK)],
                            acc_sh.at[pl.ds(slab + r * _CHUNK, _CHUNK)])
        plsc.subcore_barrier()

        def edge_body(i, carry):
            base_c = i * _K
            cps = []
            for t in range(_K):
                cps.append(pltpu.async_copy(
                    f1_hbm.at[src_v.at[base_c + t]],
                    rows_v.at[pl.ds(t * _CHUNK, _CHUNK)], sem.at[t]))
            for t in range(_K):
                cps[t].wait()
                pltpu.sync_copy(rows_v.at[pl.ds(t * _CHUNK, _CHUNK)],
                                acc_sh.at[dst_v.at[base_c + t]], add=True)
            return carry
        lax.fori_loop(0, _NCHUNK // _K, edge_body, 0)
        plsc.subcore_barrier()

        @pl.when(s < 10)
        def _():
            pltpu.sync_copy(acc_sh.at[pl.ds(s * 1000, 1000)],
                            out_hbm.at[c, pl.ds(s * 1000, 1000)])

    return body(f1, src3, dst3)


@jax.jit
def _impl(feature, edge_index, W1, b1, W2, b2, W3, b3):
    src3 = edge_index[0].reshape(_NW, _NCHUNK, _CHUNK)
    dst3 = edge_index[1].reshape(_NW, _NCHUNK, _CHUNK)

    blk = 1000
    grid = (_N // blk,)
    f1 = pl.pallas_call(
        _mm1_body,
        grid=grid,
        in_specs=[
            pl.BlockSpec((blk, _D_IN), lambda i: (i, 0)),
            pl.BlockSpec((_D_IN, _D_HID), lambda i: (0, 0)),
        ],
        out_specs=pl.BlockSpec((blk, _D_HID), lambda i: (i, 0)),
        out_shape=jax.ShapeDtypeStruct((_N, _D_HID), jnp.float32),
    )(feature, W1)

    partials = _sc_aggregate(f1, src3, dst3)

    out = pl.pallas_call(
        _mlp_body,
        grid=grid,
        in_specs=[
            pl.BlockSpec((blk, _D_HID), lambda i: (i, 0)),
            pl.BlockSpec((blk, _D_HID), lambda i: (i, 0)),
            pl.BlockSpec((1, _D_HID), lambda i: (0, 0)),
            pl.BlockSpec((_D_HID, _D_HID), lambda i: (0, 0)),
            pl.BlockSpec((1, _D_HID), lambda i: (0, 0)),
            pl.BlockSpec((_D_HID, _D_OUT), lambda i: (0, 0)),
            pl.BlockSpec((1, _D_OUT), lambda i: (0, 0)),
        ],
        out_specs=pl.BlockSpec((blk, _D_OUT), lambda i: (i, 0)),
        out_shape=jax.ShapeDtypeStruct((_N, _D_OUT), jnp.float32),
    )(partials[0], partials[1], b1.reshape(1, _D_HID), W2,
      b2.reshape(1, _D_HID), W3, b3.reshape(1, _D_OUT))
    return out


def kernel(feature, edge_index, W1, b1, W2, b2, W3, b3):
    return _impl(feature, edge_index, W1, b1, W2, b2, W3, b3)


# speedup vs baseline: 13.2258x; 1.0253x over previous
import functools

import jax
import jax.numpy as jnp
from jax import lax
from jax.experimental import pallas as pl
from jax.experimental.pallas import tpu as pltpu
from jax.experimental.pallas import tpu_sc as plsc

_N = 10000
_E = 320000
_D_IN = 128
_D_HID = 64
_D_OUT = 128

_NC = 2
_NS = 16
_NW = _NC * _NS
_EPW = _E // _NW
_CHUNK = 125
_NCHUNK = _EPW // _CHUNK
_ROWS_PT = _N // _NS
_K = 8


def _mm1_body(x_ref, w_ref, o_ref):
    o_ref[...] = jnp.dot(x_ref[...], w_ref[...], preferred_element_type=jnp.float32)


def _mlp_body(p0_ref, p1_ref, b1_ref, w2_ref, b2_ref, w3_ref, b3_ref, o_ref):
    h = jnp.maximum(p0_ref[...] + p1_ref[...] + b1_ref[...], 0.0)
    h = jnp.maximum(
        jnp.dot(h, w2_ref[...], preferred_element_type=jnp.float32) + b2_ref[...], 0.0)
    o_ref[...] = jnp.dot(h, w3_ref[...], preferred_element_type=jnp.float32) + b3_ref[...]


def _sc_aggregate(f1, src3, dst3):
    mesh = plsc.VectorSubcoreMesh(core_axis_name="c", subcore_axis_name="s")

    @functools.partial(
        pl.kernel,
        out_type=jax.ShapeDtypeStruct((_NC, _N, _D_HID), jnp.float32),
        mesh=mesh,
        scratch_types=[
            pltpu.VMEM((_NCHUNK, _CHUNK), jnp.int32),
            pltpu.VMEM((_NCHUNK, _CHUNK), jnp.int32),
            pltpu.VMEM((_K * _CHUNK, _D_HID), jnp.float32),
            pltpu.VMEM_SHARED((_N, _D_HID), jnp.float32),
            pltpu.SemaphoreType.DMA((_K,)),
            pltpu.SemaphoreType.DMA((_K,)),
        ],
        compiler_params=pltpu.CompilerParams(use_tc_tiling_on_sc=False),
    )
    def body(f1_hbm, src_hbm, dst_hbm, out_hbm, src_v, dst_v, rows_v, acc_sh,
             gsem, ssem):
        c = lax.axis_index("c")
        s = lax.axis_index("s")
        wid = s * _NC + c

        pltpu.sync_copy(src_hbm.at[wid], src_v)
        pltpu.sync_copy(dst_hbm.at[wid], dst_v)

        def zero_body(i, carry):
            for k in range(_D_HID // 16):
                rows_v[i, pl.ds(k * 16, 16)] = jnp.zeros((16,), jnp.float32)
            return carry
        lax.fori_loop(0, _CHUNK, zero_body, 0)
        slab = s * _ROWS_PT
        for r in range(_ROWS_PT // _CHUNK):
            pltpu.sync_copy(rows_v.at[pl.ds(0, _CHUNK)],
                            acc_sh.at[pl.ds(slab + r * _CHUNK, _CHUNK)])
        plsc.subcore_barrier()

        def edge_body(i, carry):
            base_c = i * _K
            gcps = []
            for t in range(_K):
                gcps.append(pltpu.async_copy(
                    f1_hbm.at[src_v.at[base_c + t]],
                    rows_v.at[pl.ds(t * _CHUNK, _CHUNK)], gsem.at[t]))
            scps = []
            for t in range(_K):
                gcps[t].wait()
                scps.append(pltpu.async_copy(
                    rows_v.at[pl.ds(t * _CHUNK, _CHUNK)],
                    acc_sh.at[dst_v.at[base_c + t]], ssem.at[t], add=True))
            for t in range(_K):
                scps[t].wait()
            return carry
        lax.fori_loop(0, _NCHUNK // _K, edge_body, 0)
        plsc.subcore_barrier()

        @pl.when(s < 10)
        def _():
            pltpu.sync_copy(acc_sh.at[pl.ds(s * 1000, 1000)],
                            out_hbm.at[c, pl.ds(s * 1000, 1000)])

    return body(f1, src3, dst3)


@jax.jit
def _impl(feature, edge_index, W1, b1, W2, b2, W3, b3):
    src3 = edge_index[0].reshape(_NW, _NCHUNK, _CHUNK)
    dst3 = edge_index[1].reshape(_NW, _NCHUNK, _CHUNK)

    blk = 1000
    grid = (_N // blk,)
    f1 = pl.pallas_call(
        _mm1_body,
        grid=grid,
        in_specs=[
            pl.BlockSpec((blk, _D_IN), lambda i: (i, 0)),
            pl.BlockSpec((_D_IN, _D_HID), lambda i: (0, 0)),
        ],
        out_specs=pl.BlockSpec((blk, _D_HID), lambda i: (i, 0)),
        out_shape=jax.ShapeDtypeStruct((_N, _D_HID), jnp.float32),
    )(feature, W1)

    partials = _sc_aggregate(f1, src3, dst3)

    out = pl.pallas_call(
        _mlp_body,
        grid=grid,
        in_specs=[
            pl.BlockSpec((blk, _D_HID), lambda i: (i, 0)),
            pl.BlockSpec((blk, _D_HID), lambda i: (i, 0)),
            pl.BlockSpec((1, _D_HID), lambda i: (0, 0)),
            pl.BlockSpec((_D_HID, _D_HID), lambda i: (0, 0)),
            pl.BlockSpec((1, _D_HID), lambda i: (0, 0)),
            pl.BlockSpec((_D_HID, _D_OUT), lambda i: (0, 0)),
            pl.BlockSpec((1, _D_OUT), lambda i: (0, 0)),
        ],
        out_specs=pl.BlockSpec((blk, _D_OUT), lambda i: (i, 0)),
        out_shape=jax.ShapeDtypeStruct((_N, _D_OUT), jnp.float32),
    )(partials[0], partials[1], b1.reshape(1, _D_HID), W2,
      b2.reshape(1, _D_HID), W3, b3.reshape(1, _D_OUT))
    return out


def kernel(feature, edge_index, W1, b1, W2, b2, W3, b3):
    return _impl(feature, edge_index, W1, b1, W2, b2, W3, b3)


# speedup vs baseline: 16.2352x; 1.2275x over previous
import functools

import jax
import jax.numpy as jnp
from jax import lax
from jax.experimental import pallas as pl
from jax.experimental.pallas import tpu as pltpu
from jax.experimental.pallas import tpu_sc as plsc

_N = 10000
_E = 320000
_D_IN = 128
_D_HID = 64
_D_OUT = 128

_NC = 2
_NS = 16
_NW = _NC * _NS
_EPW = _E // _NW
_CHUNK = 125
_NCHUNK = _EPW // _CHUNK
_K = 8


def _mm1_body(x_ref, w_ref, o_ref):
    o_ref[...] = jnp.dot(x_ref[...], w_ref[...], preferred_element_type=jnp.float32)


def _mlp_body(pa_ref, pb_ref, b1_ref, w2_ref, b2_ref, w3_ref, b3_ref, o_ref):
    h = jnp.maximum(pa_ref[...] + pb_ref[...] + b1_ref[...], 0.0)
    h = jnp.maximum(
        jnp.dot(h, w2_ref[...], preferred_element_type=jnp.float32) + b2_ref[...], 0.0)
    o_ref[...] = jnp.dot(h, w3_ref[...], preferred_element_type=jnp.float32) + b3_ref[...]


def _blockdiag2(w):
    fi, fo = w.shape
    z = jnp.zeros((fi, fo), w.dtype)
    return jnp.concatenate(
        [jnp.concatenate([w, z], axis=1), jnp.concatenate([z, w], axis=1)], axis=0)


def _sc_aggregate(f1, edge3):
    mesh = plsc.VectorSubcoreMesh(core_axis_name="c", subcore_axis_name="s")

    @functools.partial(
        pl.kernel,
        out_type=jax.ShapeDtypeStruct((_NC, _N, _D_HID), jnp.float32),
        mesh=mesh,
        scratch_types=[
            pltpu.VMEM((_NCHUNK, _CHUNK), jnp.int32),
            pltpu.VMEM((_NCHUNK, _CHUNK), jnp.int32),
            pltpu.VMEM((_K * _CHUNK, _D_HID), jnp.float32),
            pltpu.VMEM_SHARED((_N, _D_HID), jnp.float32),
            pltpu.SemaphoreType.DMA((_K,)),
            pltpu.SemaphoreType.DMA((_K,)),
        ],
        compiler_params=pltpu.CompilerParams(use_tc_tiling_on_sc=False),
    )
    def body(f1_hbm, edge_hbm, out_hbm, src_v, dst_v, rows_v, acc_sh,
             gsem, ssem):
        c = lax.axis_index("c")
        s = lax.axis_index("s")
        wid = s * _NC + c

        pltpu.sync_copy(edge_hbm.at[0, wid], src_v)
        pltpu.sync_copy(edge_hbm.at[1, wid], dst_v)

        def zero_body(i, carry):
            for k in range(_D_HID // 16):
                rows_v[i, pl.ds(k * 16, 16)] = jnp.zeros((16,), jnp.float32)
            return carry
        lax.fori_loop(0, _CHUNK, zero_body, 0)
        slab = s * (_N // _NS)
        for r in range((_N // _NS) // _CHUNK):
            pltpu.sync_copy(rows_v.at[pl.ds(0, _CHUNK)],
                            acc_sh.at[pl.ds(slab + r * _CHUNK, _CHUNK)])
        plsc.subcore_barrier()

        def edge_body(i, carry):
            base_c = i * _K
            gcps = []
            for t in range(_K):
                gcps.append(pltpu.async_copy(
                    f1_hbm.at[src_v.at[base_c + t]],
                    rows_v.at[pl.ds(t * _CHUNK, _CHUNK)], gsem.at[t]))
            scps = []
            for t in range(_K):
                gcps[t].wait()
                scps.append(pltpu.async_copy(
                    rows_v.at[pl.ds(t * _CHUNK, _CHUNK)],
                    acc_sh.at[dst_v.at[base_c + t]], ssem.at[t], add=True))
            for t in range(_K):
                scps[t].wait()
            return carry
        lax.fori_loop(0, _NCHUNK // _K, edge_body, 0)
        plsc.subcore_barrier()

        @pl.when(s < 10)
        def _():
            pltpu.sync_copy(acc_sh.at[pl.ds(s * 1000, 1000)],
                            out_hbm.at[c, pl.ds(s * 1000, 1000)])

    return body(f1, edge3)


@jax.jit
def _impl(feature, edge_index, W1, b1, W2, b2, W3, b3):
    edge3 = edge_index.reshape(2, _NW, _NCHUNK, _CHUNK)
    featp = feature.reshape(_N // 2, 2 * _D_IN)

    nb = 5
    blk = _N // 2 // nb
    grid = (nb,)
    f1p = pl.pallas_call(
        _mm1_body,
        grid=grid,
        in_specs=[
            pl.BlockSpec((blk, 2 * _D_IN), lambda i: (i, 0)),
            pl.BlockSpec((2 * _D_IN, 2 * _D_HID), lambda i: (0, 0)),
        ],
        out_specs=pl.BlockSpec((blk, 2 * _D_HID), lambda i: (i, 0)),
        out_shape=jax.ShapeDtypeStruct((_N // 2, 2 * _D_HID), jnp.float32),
    )(featp, _blockdiag2(W1))
    f1 = f1p.reshape(_N, _D_HID)

    partials = _sc_aggregate(f1, edge3)
    p128 = partials.reshape(_N, 2 * _D_HID)

    outp = pl.pallas_call(
        _mlp_body,
        grid=grid,
        in_specs=[
            pl.BlockSpec((blk, 2 * _D_HID), lambda i: (i, 0)),
            pl.BlockSpec((blk, 2 * _D_HID), lambda i: (i + nb, 0)),
            pl.BlockSpec((1, 2 * _D_HID), lambda i: (0, 0)),
            pl.BlockSpec((2 * _D_HID, 2 * _D_HID), lambda i: (0, 0)),
            pl.BlockSpec((1, 2 * _D_HID), lambda i: (0, 0)),
            pl.BlockSpec((2 * _D_HID, 2 * _D_OUT), lambda i: (0, 0)),
            pl.BlockSpec((1, 2 * _D_OUT), lambda i: (0, 0)),
        ],
        out_specs=pl.BlockSpec((blk, 2 * _D_OUT), lambda i: (i, 0)),
        out_shape=jax.ShapeDtypeStruct((_N // 2, 2 * _D_OUT), jnp.float32),
    )(p128, p128,
      jnp.tile(b1, 2).reshape(1, 2 * _D_HID), _blockdiag2(W2),
      jnp.tile(b2, 2).reshape(1, 2 * _D_HID), _blockdiag2(W3),
      jnp.tile(b3, 2).reshape(1, 2 * _D_OUT))
    return outp.reshape(_N, _D_OUT)


def kernel(feature, edge_index, W1, b1, W2, b2, W3, b3):
    return _impl(feature, edge_index, W1, b1, W2, b2, W3, b3)
